# Initial kernel scaffold; baseline (speedup 1.0000x reference)
import jax
import jax.numpy as jnp
from jax.experimental import pallas as pl


def kernel(target_wids, context_wids, neg_wids, i_emb, o_emb, W1, b1, W2, b2):
    raise NotImplementedError("write your pallas kernel here")


# speedup vs baseline: 2.2022x; 2.2022x over previous
import functools

import jax
import jax.numpy as jnp
from jax import lax
from jax.experimental import pallas as pl
from jax.experimental.pallas import tpu as pltpu
from jax.experimental.pallas import tpu_sc as plsc

D = 64
CTX = 10
NNEG = 20

NC = 2
NS = 16
NW = NC * NS
CHUNK = 512


def _sc_gather(i_emb, o_emb, ctx_idx, tn_idx):
  n_ctx = ctx_idx.shape[0]
  n_tn = tn_idx.shape[0]
  ctx_pw = n_ctx // NW
  tn_pw = n_tn // NW

  mesh = plsc.VectorSubcoreMesh(core_axis_name="c", subcore_axis_name="s")

  @functools.partial(
      pl.kernel,
      mesh=mesh,
      out_type=[
          jax.ShapeDtypeStruct((n_ctx, D), jnp.float32),
          jax.ShapeDtypeStruct((n_tn, D), jnp.float32),
      ],
      scratch_types=[
          pltpu.VMEM((CHUNK,), jnp.int32),
          pltpu.VMEM((CHUNK, D), jnp.float32),
          pltpu.SemaphoreType.DMA,
      ],
      compiler_params=pltpu.CompilerParams(use_tc_tiling_on_sc=False),
  )
  def gather_k(i_emb_h, o_emb_h, ctx_idx_h, tn_idx_h, ctx_out, tn_out,
               idx_v, rows_v, sem):
    wid = lax.axis_index("s") * NC + lax.axis_index("c")

    def run(idx_h, tab_h, out_h, per_worker):
      base = wid * per_worker

      def body(i, carry):
        b = pl.multiple_of(base + i * CHUNK, CHUNK)
        pltpu.sync_copy(idx_h.at[pl.ds(b, CHUNK)], idx_v)
        pltpu.async_copy(tab_h.at[idx_v], rows_v, sem).wait()
        pltpu.sync_copy(rows_v, out_h.at[pl.ds(b, CHUNK)])
        return carry

      lax.fori_loop(0, per_worker // CHUNK, body, 0)

    run(ctx_idx_h, i_emb_h, ctx_out, ctx_pw)
    run(tn_idx_h, o_emb_h, tn_out, tn_pw)

  return gather_k(i_emb, o_emb, ctx_idx, tn_idx)


def _dense_body(ctx_ref, tgt_ref, neg_ref, W1_ref, b1_ref, W2_ref, b2_ref,
                out_ref):
  ctx = ctx_ref[...]
  h = jnp.tanh(
      jnp.dot(ctx, W1_ref[...], preferred_element_type=jnp.float32)
      + b1_ref[...])
  logits = jnp.dot(h, W2_ref[...],
                   preferred_element_type=jnp.float32) + b2_ref[...]
  a = jax.nn.softmax(logits, axis=-1)

  attn = a[:, 0:1] * ctx[:, 0:D]
  for j in range(1, CTX):
    attn = attn + a[:, j:j + 1] * ctx[:, j * D:(j + 1) * D]

  pos_dot = jnp.sum(tgt_ref[...] * attn, axis=1)
  acc = jnp.sum(jnp.log(jax.nn.sigmoid(pos_dot)))

  neg = neg_ref[...]
  for j in range(NNEG):
    nd = jnp.sum(neg[:, j * D:(j + 1) * D] * attn, axis=1)
    acc = acc + jnp.sum(jnp.log(jax.nn.sigmoid(-nd)))

  @pl.when(pl.program_id(0) == 0)
  def _():
    out_ref[0, 0] = 0.0

  out_ref[0, 0] += acc


def kernel(target_wids, context_wids, neg_wids, i_emb, o_emb, W1, b1, W2, b2):
  B = target_wids.shape[0]
  ctx_idx = context_wids.reshape(-1).astype(jnp.int32)
  tn_idx = jnp.concatenate(
      [target_wids.astype(jnp.int32), neg_wids.reshape(-1).astype(jnp.int32)])

  ctx_rows, tn_rows = _sc_gather(i_emb, o_emb, ctx_idx, tn_idx)
  ctx_flat = ctx_rows.reshape(B, CTX * D)
  tgt = tn_rows[:B]
  neg_flat = tn_rows[B:].reshape(B, NNEG * D)

  bB = 1024
  grid = B // bB
  loss = pl.pallas_call(
      _dense_body,
      grid=(grid,),
      in_specs=[
          pl.BlockSpec((bB, CTX * D), lambda i: (i, 0)),
          pl.BlockSpec((bB, D), lambda i: (i, 0)),
          pl.BlockSpec((bB, NNEG * D), lambda i: (i, 0)),
          pl.BlockSpec((CTX * D, 50), lambda i: (0, 0)),
          pl.BlockSpec((1, 50), lambda i: (0, 0)),
          pl.BlockSpec((50, CTX), lambda i: (0, 0)),
          pl.BlockSpec((1, CTX), lambda i: (0, 0)),
      ],
      out_specs=pl.BlockSpec((1, 1), lambda i: (0, 0),
                             memory_space=pltpu.SMEM),
      out_shape=jax.ShapeDtypeStruct((1, 1), jnp.float32),
  )(ctx_flat, tgt, neg_flat, W1, b1.reshape(1, 50), W2, b2.reshape(1, CTX))

  return -loss[0, 0]


# speedup vs baseline: 3.4024x; 1.5450x over previous
import functools

import jax
import jax.numpy as jnp
from jax import lax
from jax.experimental import pallas as pl
from jax.experimental.pallas import tpu as pltpu
from jax.experimental.pallas import tpu_sc as plsc

D = 64
CTX = 10
NNEG = 20

NC = 2
NS = 16
NW = NC * NS
CHUNK = 512


def _pipelined_gather(tab_h, idx_h, n_chunks, base, out_h, ibufs, bufs, sems):
  i0, i1 = ibufs
  buf0, buf1 = bufs
  sem0, sem1 = sems

  def load_idx(c, ibuf):
    b = pl.multiple_of(base + c * CHUNK, CHUNK)
    pltpu.sync_copy(idx_h.at[pl.ds(b, CHUNK)], ibuf)

  def start(ibuf, buf, sem):
    pltpu.async_copy(tab_h.at[ibuf], buf, sem)

  def wait(ibuf, buf, sem):
    pltpu.make_async_copy(tab_h.at[ibuf], buf, sem).wait()

  def scatter(c, buf):
    b = pl.multiple_of(base + c * CHUNK, CHUNK)
    pltpu.sync_copy(buf, out_h.at[pl.ds(b, CHUNK)])

  if n_chunks == 1:
    load_idx(0, i0)
    start(i0, buf0, sem0)
    wait(i0, buf0, sem0)
    scatter(0, buf0)
    return

  nh = n_chunks // 2
  load_idx(0, i0)
  start(i0, buf0, sem0)

  def body(j, carry):
    c0 = j * 2
    load_idx(c0 + 1, i1)
    start(i1, buf1, sem1)
    wait(i0, buf0, sem0)
    scatter(c0, buf0)

    @pl.when(j < nh - 1)
    def _():
      load_idx(c0 + 2, i0)
      start(i0, buf0, sem0)

    wait(i1, buf1, sem1)
    scatter(c0 + 1, buf1)
    return carry

  lax.fori_loop(0, nh, body, 0)


def _sc_gather(i_emb, o_emb, ctx_idx, tgt_idx, neg_idx):
  nc_ctx = ctx_idx.shape[0] // (NW * CHUNK)
  nc_tgt = tgt_idx.shape[0] // (NW * CHUNK)
  nc_neg = neg_idx.shape[0] // (NW * CHUNK)

  mesh = plsc.VectorSubcoreMesh(core_axis_name="c", subcore_axis_name="s")

  @functools.partial(
      pl.kernel,
      mesh=mesh,
      out_type=[
          jax.ShapeDtypeStruct((NW * nc_ctx * CHUNK, D), jnp.float32),
          jax.ShapeDtypeStruct((NW * nc_tgt * CHUNK, D), jnp.float32),
          jax.ShapeDtypeStruct((NW * nc_neg * CHUNK, D), jnp.float32),
      ],
      scratch_types=[
          pltpu.VMEM((CHUNK,), jnp.int32),
          pltpu.VMEM((CHUNK,), jnp.int32),
          pltpu.VMEM((CHUNK, D), jnp.float32),
          pltpu.VMEM((CHUNK, D), jnp.float32),
          pltpu.SemaphoreType.DMA,
          pltpu.SemaphoreType.DMA,
      ],
      compiler_params=pltpu.CompilerParams(use_tc_tiling_on_sc=False),
  )
  def gather_k(i_emb_h, o_emb_h, ctx_idx_h, tgt_idx_h, neg_idx_h,
               ctx_out, tgt_out, neg_out,
               i0, i1, buf0, buf1, sem0, sem1):
    wid = lax.axis_index("s") * NC + lax.axis_index("c")
    ibufs = (i0, i1)
    bufs = (buf0, buf1)
    sems = (sem0, sem1)
    _pipelined_gather(i_emb_h, ctx_idx_h, nc_ctx, wid * (nc_ctx * CHUNK),
                      ctx_out, ibufs, bufs, sems)
    _pipelined_gather(o_emb_h, tgt_idx_h, nc_tgt, wid * (nc_tgt * CHUNK),
                      tgt_out, ibufs, bufs, sems)
    _pipelined_gather(o_emb_h, neg_idx_h, nc_neg, wid * (nc_neg * CHUNK),
                      neg_out, ibufs, bufs, sems)

  return gather_k(i_emb, o_emb, ctx_idx, tgt_idx, neg_idx)


def _dense_body(ctx_ref, tgt_ref, neg_ref, W1_ref, b1_ref, W2_ref, b2_ref,
                out_ref):
  ctx = ctx_ref[...]
  h = jnp.tanh(
      jnp.dot(ctx, W1_ref[...], preferred_element_type=jnp.float32)
      + b1_ref[...])
  logits = jnp.dot(h, W2_ref[...],
                   preferred_element_type=jnp.float32) + b2_ref[...]
  a = jax.nn.softmax(logits, axis=-1)

  attn = a[:, 0:1] * ctx[:, 0:D]
  for j in range(1, CTX):
    attn = attn + a[:, j:j + 1] * ctx[:, j * D:(j + 1) * D]

  pos_dot = jnp.sum(tgt_ref[...] * attn, axis=1)
  acc = jnp.sum(jnp.log(jax.nn.sigmoid(pos_dot)))

  neg = neg_ref[...]
  for j in range(NNEG):
    nd = jnp.sum(neg[:, j * D:(j + 1) * D] * attn, axis=1)
    acc = acc + jnp.sum(jnp.log(jax.nn.sigmoid(-nd)))

  @pl.when(pl.program_id(0) == 0)
  def _():
    out_ref[0, 0] = 0.0

  out_ref[0, 0] += acc


def kernel(target_wids, context_wids, neg_wids, i_emb, o_emb, W1, b1, W2, b2):
  B = target_wids.shape[0]
  ctx_idx = context_wids.astype(jnp.int32).reshape(-1)
  tgt_idx = target_wids.astype(jnp.int32).reshape(-1)
  neg_idx = neg_wids.astype(jnp.int32).reshape(-1)

  ctx_rows, tgt_rows, neg_rows = _sc_gather(i_emb, o_emb, ctx_idx, tgt_idx,
                                            neg_idx)
  ctx_flat = ctx_rows.reshape(B, CTX * D)
  tgt = tgt_rows
  neg_flat = neg_rows.reshape(B, NNEG * D)

  bB = 1024
  grid = B // bB
  loss = pl.pallas_call(
      _dense_body,
      grid=(grid,),
      in_specs=[
          pl.BlockSpec((bB, CTX * D), lambda i: (i, 0)),
          pl.BlockSpec((bB, D), lambda i: (i, 0)),
          pl.BlockSpec((bB, NNEG * D), lambda i: (i, 0)),
          pl.BlockSpec((CTX * D, 50), lambda i: (0, 0)),
          pl.BlockSpec((1, 50), lambda i: (0, 0)),
          pl.BlockSpec((50, CTX), lambda i: (0, 0)),
          pl.BlockSpec((1, CTX), lambda i: (0, 0)),
      ],
      out_specs=pl.BlockSpec((1, 1), lambda i: (0, 0),
                             memory_space=pltpu.SMEM),
      out_shape=jax.ShapeDtypeStruct((1, 1), jnp.float32),
  )(ctx_flat, tgt, neg_flat, W1, b1.reshape(1, 50), W2, b2.reshape(1, CTX))

  return -loss[0, 0]
